# baseline (speedup 1.0000x reference)
import jax
import jax.numpy as jnp
from jax.experimental import pallas as pl
from jax.experimental.pallas import tpu as pltpu

S = 2048
M = 1024
H = 2048
E = 8
CAP = 2 * S // E

HB = 2048
NH = H // HB


def _fused_kernel(x_ref, wg_ref, wi_ref, wo_ref, out_ref,
                  gates_scr, a_scr, p_scr, b_scr):
    t = pl.program_id(0)
    e = t // 2

    @pl.when(t == 0)
    def _():
        x = x_ref[...]
        g = jnp.dot(x, wg_ref[...], preferred_element_type=jnp.float32)
        gates_scr[...] = g
        a_scr[...] = jnp.dot(g.T, x, preferred_element_type=jnp.float32)
        p_scr[...] = jnp.zeros((E, M), jnp.float32)

    @pl.when(t % 2 == 0)
    def _():
        b = jnp.dot(a_scr[...], wi_ref[0],
                    preferred_element_type=jnp.float32)
        b = jnp.maximum(b, 0.0)
        onehot = (jax.lax.broadcasted_iota(jnp.int32, (E, 1), 0) == e)
        b_scr[...] = jnp.where(onehot, b, 0.0)

    @pl.when(t % 2 == 1)
    def _():
        p_scr[...] += jnp.dot(b_scr[...], wo_ref[0],
                              preferred_element_type=jnp.float32)

    @pl.when(t == 2 * E - 1)
    def _():
        out_ref[...] = jnp.dot(
            gates_scr[...] * float(CAP), p_scr[...],
            preferred_element_type=jnp.float32)


def kernel(inputs, wg, wi, wo):
    x = jnp.reshape(jnp.asarray(inputs, jnp.float32), (S, M))

    out = pl.pallas_call(
        _fused_kernel,
        grid=(2 * E,),
        in_specs=[
            pl.BlockSpec((S, M), lambda t: (0, 0)),
            pl.BlockSpec((M, E), lambda t: (0, 0)),
            pl.BlockSpec((1, M, HB), lambda t: (t // 2, 0, 0)),
            pl.BlockSpec((1, HB, M),
                         lambda t: (jnp.maximum((t - 1) // 2, 0), 0, 0)),
        ],
        out_specs=pl.BlockSpec((S, M), lambda t: (0, 0)),
        out_shape=jax.ShapeDtypeStruct((S, M), jnp.float32),
        scratch_shapes=[
            pltpu.VMEM((S, E), jnp.float32),
            pltpu.VMEM((E, M), jnp.float32),
            pltpu.VMEM((E, M), jnp.float32),
            pltpu.VMEM((E, H), jnp.float32),
        ],
    )(x, wg, wi, wo)

    return jnp.reshape(out, inputs.shape)


# speedup vs baseline: 1.0520x; 1.0520x over previous
import jax
import jax.numpy as jnp
from jax.experimental import pallas as pl
from jax.experimental.pallas import tpu as pltpu

S = 2048
M = 1024
H = 2048
E = 8
CAP = 2 * S // E

HB = 2048
NH = H // HB


def _fused_kernel(x_ref, wg_ref, wi_ref, wo_ref, out_ref,
                  gates_scr, a_scr, p_scr):
    e = pl.program_id(0)

    @pl.when(e == 0)
    def _():
        x = x_ref[...]
        g = jnp.dot(x, wg_ref[...], preferred_element_type=jnp.float32)
        gates_scr[...] = g
        a_scr[...] = jnp.dot(g.T, x, preferred_element_type=jnp.float32)
        p_scr[...] = jnp.zeros((E, M), jnp.float32)

    b = jnp.dot(a_scr[...], wi_ref[0], preferred_element_type=jnp.float32)
    b = jnp.maximum(b, 0.0)
    onehot = (jax.lax.broadcasted_iota(jnp.int32, (E, 1), 0) == e)
    b = jnp.where(onehot, b, 0.0)
    p_scr[...] += jnp.dot(b, wo_ref[0], preferred_element_type=jnp.float32)

    @pl.when(e == E - 1)
    def _():
        out_ref[...] = jnp.dot(
            gates_scr[...] * float(CAP), p_scr[...],
            preferred_element_type=jnp.float32)


def kernel(inputs, wg, wi, wo):
    x = jnp.reshape(jnp.asarray(inputs, jnp.float32), (S, M))

    out = pl.pallas_call(
        _fused_kernel,
        grid=(E,),
        in_specs=[
            pl.BlockSpec((S, M), lambda e: (0, 0)),
            pl.BlockSpec((M, E), lambda e: (0, 0)),
            pl.BlockSpec((1, M, HB), lambda e: (e, 0, 0)),
            pl.BlockSpec((1, HB, M), lambda e: (e, 0, 0)),
        ],
        out_specs=pl.BlockSpec((S, M), lambda e: (0, 0)),
        out_shape=jax.ShapeDtypeStruct((S, M), jnp.float32),
        scratch_shapes=[
            pltpu.VMEM((S, E), jnp.float32),
            pltpu.VMEM((E, M), jnp.float32),
            pltpu.VMEM((E, M), jnp.float32),
        ],
    )(x, wg, wi, wo)

    return jnp.reshape(out, inputs.shape)
